# baseline (speedup 1.0000x reference)
import functools

import jax
import jax.numpy as jnp
from jax import lax
from jax.experimental import pallas as pl
from jax.experimental.pallas import tpu as pltpu
from jax.experimental.pallas import tpu_sc as plsc

N0 = 10000
N0P = 10240
N1 = 160000
EB = 128
SB = 2048
NT = 16
NC = 2

NR0 = -(-330000 // (NT * EB))
NR1 = -(-320000 // (NT * EB))
NBL1 = -(-800000 // (NT * SB))
NBB1 = -(-320000 // (NT * SB))


def _comb_body(pair, x1_ref, x2_ref, x3_ref, w1_ref, w2_ref, w3_ref, o_ref):
    x1 = x1_ref[...]
    if pair:
        x1, x2 = x1[0], x1[1]
    else:
        x2 = x2_ref[...]
    m = (jnp.dot(x1, w1_ref[...], preferred_element_type=jnp.float32)
         + jnp.dot(x2, w2_ref[...], preferred_element_type=jnp.float32)
         + jnp.dot(x3_ref[...], w3_ref[...], preferred_element_type=jnp.float32))
    o_ref[...] = jnp.tanh(m)


def _combine(x1, x2, x3, w1, w2, w3, pair, blk):
    n, d = x3.shape
    if pair:
        s1 = pl.BlockSpec((2, blk, d), lambda i: (0, i, 0))
        s2 = pl.BlockSpec((1, 1), lambda i: (0, 0))
        x2 = jnp.zeros((1, 1), jnp.float32)
    else:
        s1 = pl.BlockSpec((blk, d), lambda i: (i, 0))
        s2 = pl.BlockSpec((blk, d), lambda i: (i, 0))
    dout = w1.shape[1]
    wspec = pl.BlockSpec((d, dout), lambda i: (0, 0))
    return pl.pallas_call(
        functools.partial(_comb_body, pair),
        grid=(n // blk,),
        in_specs=[s1, s2, pl.BlockSpec((blk, d), lambda i: (i, 0)),
                  wspec, wspec, wspec],
        out_specs=pl.BlockSpec((blk, dout), lambda i: (i, 0)),
        out_shape=jax.ShapeDtypeStruct((n, dout), jnp.float32),
    )(x1, x2, x3, w1, w2, w3)


def _fc_body(x_ref, w_ref, o_ref):
    o_ref[...] = jnp.dot(x_ref[...], w_ref[...], preferred_element_type=jnp.float32)


def _fc(x, w, blk):
    n, d = x.shape
    dout = w.shape[1]
    return pl.pallas_call(
        _fc_body,
        grid=(n // blk,),
        in_specs=[pl.BlockSpec((blk, d), lambda i: (i, 0)),
                  pl.BlockSpec((d, dout), lambda i: (0, 0))],
        out_specs=pl.BlockSpec((blk, dout), lambda i: (i, 0)),
        out_shape=jax.ShapeDtypeStruct((n, dout), jnp.float32),
    )(x, w)


def _scale_rows(rows_v, val_ref, koff, W):
    def grp(g, carry):
        vv = val_ref[pl.ds(koff + 16 * g, 16)]
        for e in range(16):
            splat = jnp.broadcast_to(vv[e], (16,))
            r = 16 * g + e
            for f in range(W // 16):
                rows_v[r, pl.ds(16 * f, 16)] = rows_v[r, pl.ds(16 * f, 16)] * splat
        return carry

    lax.fori_loop(0, EB // 16, grp, 0)


@functools.cache
def _make_spmm0(W):
    mesh = plsc.VectorSubcoreMesh(core_axis_name="c", subcore_axis_name="s")
    stripe = N0P // NT

    @functools.partial(
        pl.kernel,
        out_type=jax.ShapeDtypeStruct((NC, N0P, W), jnp.float32),
        mesh=mesh,
        compiler_params=pltpu.CompilerParams(use_tc_tiling_on_sc=False,
                                             needs_layout_passes=False),
        scratch_types=[
            pltpu.VMEM_SHARED((N0P, W), jnp.float32),
            pltpu.VMEM((EB,), jnp.int32),
            pltpu.VMEM((EB,), jnp.int32),
            pltpu.VMEM((EB,), jnp.float32),
            pltpu.VMEM((EB, W), jnp.float32),
            pltpu.VMEM((8, W), jnp.float32),
            pltpu.SemaphoreType.DMA,
        ],
    )
    def spmm0(h0_hbm, h1_hbm, e0d, e0s, e0v, e1d, e1s, e1v,
              p_hbm, acc, dst_v, src_v, val_v, rows_v, zb, sem):
        c = lax.axis_index("c")
        s = lax.axis_index("s")
        base = s * stripe

        z = jnp.zeros((16,), jnp.float32)
        for r in range(8):
            for i in range(W // 16):
                zb[r, pl.ds(16 * i, 16)] = z

        def zrow(i, carry):
            pltpu.sync_copy(zb, acc.at[pl.ds(base + i * 8, 8)])
            return carry
        lax.fori_loop(0, stripe // 8, zrow, 0)
        plsc.subcore_barrier()

        def process(d1, s1, v1, src_hbm, nrows):
            def body(j, carry):
                off = pl.multiple_of((s * nrows + j) * EB, 8)
                pltpu.sync_copy(d1.at[pl.ds(off, EB)], dst_v)
                pltpu.sync_copy(s1.at[pl.ds(off, EB)], src_v)
                pltpu.sync_copy(v1.at[pl.ds(off, EB)], val_v)
                pltpu.async_copy(src_hbm.at[src_v], rows_v, sem).wait()
                _scale_rows(rows_v, val_v, 0, W)
                pltpu.sync_copy(rows_v, acc.at[dst_v], add=True)
                return carry

            lax.fori_loop(0, nrows, body, 0)

        @pl.when(c == 0)
        def _():
            process(e0d, e0s, e0v, h0_hbm, NR0)

        @pl.when(c == 1)
        def _():
            process(e1d, e1s, e1v, h1_hbm, NR1)

        plsc.subcore_barrier()
        pltpu.sync_copy(acc.at[pl.ds(base, stripe)], p_hbm.at[c, pl.ds(base, stripe)])

    return spmm0


@functools.cache
def _make_spmm1(W, R, nblk):
    mesh = plsc.VectorSubcoreMesh(core_axis_name="c", subcore_axis_name="s")
    CAP = SB + EB + 16
    NCH = -(-N1 // R)
    stripe = R // NT

    @functools.partial(
        pl.kernel,
        out_type=jax.ShapeDtypeStruct((NCH * R, W), jnp.float32),
        mesh=mesh,
        compiler_params=pltpu.CompilerParams(use_tc_tiling_on_sc=False,
                                             needs_layout_passes=False),
        scratch_types=[
            pltpu.VMEM_SHARED((R, W), jnp.float32),
            pltpu.VMEM((SB,), jnp.int32),
            pltpu.VMEM((SB,), jnp.int32),
            pltpu.VMEM((SB,), jnp.float32),
            pltpu.VMEM((CAP,), jnp.int32),
            pltpu.VMEM((CAP,), jnp.int32),
            pltpu.VMEM((CAP,), jnp.float32),
            pltpu.VMEM((EB,), jnp.int32),
            pltpu.VMEM((EB,), jnp.int32),
            pltpu.VMEM((EB, W), jnp.float32),
            pltpu.VMEM((8, W), jnp.float32),
            pltpu.SemaphoreType.DMA,
        ],
    )
    def spmm1(src_hbm, d1, s1, v1,
              out_hbm, acc, ed, es, ev, cd, cs, cv, dst128, src128, rows_v, zb, sem):
        c = lax.axis_index("c")
        s = lax.axis_index("s")

        z = jnp.zeros((16,), jnp.float32)
        for r in range(8):
            for i in range(W // 16):
                zb[r, pl.ds(16 * i, 16)] = z

        def scan_and_process(lo):
            tile_e0 = s * nblk * SB

            def blk_body(b, carry):
                boff = pl.multiple_of(tile_e0 + b * SB, 8)
                pltpu.sync_copy(d1.at[pl.ds(boff, SB)], ed)
                pltpu.sync_copy(s1.at[pl.ds(boff, SB)], es)
                pltpu.sync_copy(v1.at[pl.ds(boff, SB)], ev)

                def grp(g, fill):
                    d16 = ed[pl.ds(16 * g, 16)]
                    loc = d16 - lo
                    m = (loc >= 0) & (loc < R)
                    cnt = plsc.all_reduce_population_count(m)[0]
                    plsc.store_compressed(cd.at[pl.ds(fill, 16)], loc, mask=m)
                    plsc.store_compressed(cs.at[pl.ds(fill, 16)], es[pl.ds(16 * g, 16)], mask=m)
                    plsc.store_compressed(cv.at[pl.ds(fill, 16)], ev[pl.ds(16 * g, 16)], mask=m)
                    return fill + cnt

                fill = lax.fori_loop(0, SB // 16, grp, jnp.int32(0))

                zi = jnp.zeros((16,), jnp.int32)
                zf = jnp.zeros((16,), jnp.float32)
                for k in range(EB // 16):
                    cd[pl.ds(fill + 16 * k, 16)] = zi
                    cs[pl.ds(fill + 16 * k, 16)] = zi
                    cv[pl.ds(fill + 16 * k, 16)] = zf

                def sub(k, carry2):
                    koff = k * EB
                    for v in range(EB // 16):
                        dst128[pl.ds(16 * v, 16)] = cd[pl.ds(koff + 16 * v, 16)]
                        src128[pl.ds(16 * v, 16)] = cs[pl.ds(koff + 16 * v, 16)]
                    pltpu.async_copy(src_hbm.at[src128], rows_v, sem).wait()
                    _scale_rows(rows_v, cv, koff, W)
                    pltpu.sync_copy(rows_v, acc.at[dst128], add=True)
                    return carry2

                lax.fori_loop(0, (fill + EB - 1) // EB, sub, 0)
                return carry

            lax.fori_loop(0, nblk, blk_body, 0)

        def chunk_body(ch, carry):
            chunk = c * (NCH // NC) + ch
            base = pl.multiple_of(chunk * R + s * stripe, 8)

            def zrow(i, carry2):
                pltpu.sync_copy(zb, acc.at[pl.ds(s * stripe + i * 8, 8)])
                return carry2
            lax.fori_loop(0, stripe // 8, zrow, 0)
            plsc.subcore_barrier()
            scan_and_process(chunk * R)
            plsc.subcore_barrier()
            pltpu.sync_copy(acc.at[pl.ds(s * stripe, stripe)], out_hbm.at[pl.ds(base, stripe)])
            plsc.subcore_barrier()
            return carry

        lax.fori_loop(0, NCH // NC, chunk_body, 0)

    return spmm1


def _pad_zero(a, total):
    return jnp.concatenate([a, jnp.zeros((total - a.shape[0],), a.dtype)])


def _pad_oob(d, s, v, total):
    pad = total - d.shape[0]
    return (jnp.concatenate([d, jnp.full((pad,), jnp.int32(1 << 29))]),
            jnp.concatenate([s, jnp.zeros((pad,), s.dtype)]),
            jnp.concatenate([v, jnp.zeros((pad,), v.dtype)]))


def kernel(X0, X1, B1_row, B1_col, B1_val, L0_row, L0_col, L0_val, L1_row, L1_col, L1_val, W1_0L, W1_0B, W1_0I, W1_1L, W1_1B, W1_1I, W2_0L, W2_0B, W2_0I, W2_1L, W2_1B, W2_1I, W3_0L, W3_0B, W3_0I, W3_1L, W3_1B, W3_1I, Wfc):
    t0 = NR0 * NT * EB
    s0 = (_pad_zero(L0_row, t0), _pad_zero(L0_col, t0), _pad_zero(L0_val, t0))
    t1 = NR1 * NT * EB
    s1 = (_pad_zero(B1_row, t1), _pad_zero(B1_col, t1), _pad_zero(B1_val, t1))
    uL = _pad_oob(L1_row, L1_col, L1_val, NBL1 * NT * SB)
    uB = _pad_oob(B1_col, B1_row, B1_val, NBB1 * NT * SB)

    Ws = [(W1_0L, W1_0B, W1_0I, W1_1L, W1_1B, W1_1I),
          (W2_0L, W2_0B, W2_0I, W2_1L, W2_1B, W2_1I),
          (W3_0L, W3_0B, W3_0I, W3_1L, W3_1B, W3_1I)]

    h0 = jnp.concatenate([X0, jnp.zeros((N0P - N0, X0.shape[1]), jnp.float32)])
    h1 = X1
    for li, (WL0, WB0, WI0, WL1, WB1, WI1) in enumerate(Ws):
        W = h0.shape[1]
        R = 8192 if W == 128 else 16384
        spmm0 = _make_spmm0(W)
        spmm1L = _make_spmm1(W, R, NBL1)
        spmm1B = _make_spmm1(W, R, NBB1)
        P = spmm0(h0, h1, *s0, *s1)
        SL = spmm1L(h1, *uL)
        SBt = spmm1B(h0, *uB)
        h0 = _combine(P, None, h0, WL0, WB0, WI0, pair=True, blk=2048)
        h1 = _combine(SL, SBt, h1, WL1, WB1, WI1, pair=False, blk=2000)

    out0 = _fc(h0, Wfc, blk=2048)[:N0]
    out1 = _fc(h1, Wfc, blk=2000)
    return (out0, out1)


# speedup vs baseline: 1.0037x; 1.0037x over previous
import functools

import jax
import jax.numpy as jnp
from jax import lax
from jax.experimental import pallas as pl
from jax.experimental.pallas import tpu as pltpu
from jax.experimental.pallas import tpu_sc as plsc

N0 = 10000
N0P = 10240
N1 = 160000
EB = 128
SB = 2048
NT = 16
NC = 2

T0 = 335872
T1 = 327680
NBL1 = -(-800000 // (NT * SB))
NBB1 = -(-320000 // (NT * SB))


def _comb_body(pair, x1_ref, x2_ref, x3_ref, w1_ref, w2_ref, w3_ref, o_ref):
    x1 = x1_ref[...]
    if pair:
        x1, x2 = x1[0], x1[1]
    else:
        x2 = x2_ref[...]
    m = (jnp.dot(x1, w1_ref[...], preferred_element_type=jnp.float32)
         + jnp.dot(x2, w2_ref[...], preferred_element_type=jnp.float32)
         + jnp.dot(x3_ref[...], w3_ref[...], preferred_element_type=jnp.float32))
    o_ref[...] = jnp.tanh(m)


def _combine(x1, x2, x3, w1, w2, w3, pair, blk):
    n, d = x3.shape
    if pair:
        s1 = pl.BlockSpec((2, blk, d), lambda i: (0, i, 0))
        s2 = pl.BlockSpec((1, 1), lambda i: (0, 0))
        x2 = jnp.zeros((1, 1), jnp.float32)
    else:
        s1 = pl.BlockSpec((blk, d), lambda i: (i, 0))
        s2 = pl.BlockSpec((blk, d), lambda i: (i, 0))
    dout = w1.shape[1]
    wspec = pl.BlockSpec((d, dout), lambda i: (0, 0))
    return pl.pallas_call(
        functools.partial(_comb_body, pair),
        grid=(n // blk,),
        in_specs=[s1, s2, pl.BlockSpec((blk, d), lambda i: (i, 0)),
                  wspec, wspec, wspec],
        out_specs=pl.BlockSpec((blk, dout), lambda i: (i, 0)),
        out_shape=jax.ShapeDtypeStruct((n, dout), jnp.float32),
    )(x1, x2, x3, w1, w2, w3)


def _fc_body(x_ref, w_ref, o_ref):
    o_ref[...] = jnp.dot(x_ref[...], w_ref[...], preferred_element_type=jnp.float32)


def _fc(x, w, blk):
    n, d = x.shape
    dout = w.shape[1]
    return pl.pallas_call(
        _fc_body,
        grid=(n // blk,),
        in_specs=[pl.BlockSpec((blk, d), lambda i: (i, 0)),
                  pl.BlockSpec((d, dout), lambda i: (0, 0))],
        out_specs=pl.BlockSpec((blk, dout), lambda i: (i, 0)),
        out_shape=jax.ShapeDtypeStruct((n, dout), jnp.float32),
    )(x, w)


def _sel(n, traced_idx, f):
    for i in range(n):
        @pl.when(traced_idx == i)
        def _(i=i):
            f(i)


def _scale_unit(rbuf, p, val_ref, voff, W, ue):
    def grp(g, carry):
        vv = val_ref[pl.ds(voff + 16 * g, 16)]
        for e in range(16):
            splat = jnp.broadcast_to(vv[e], (16,))
            r = 16 * g + e
            for f in range(W // 16):
                rbuf[p, r, pl.ds(16 * f, 16)] = rbuf[p, r, pl.ds(16 * f, 16)] * splat
        return carry

    lax.fori_loop(0, ue // 16, grp, 0)


@functools.cache
def _make_spmm0(W):
    mesh = plsc.VectorSubcoreMesh(core_axis_name="c", subcore_axis_name="s")
    stripe = N0P // NT
    ue = 256 if W == 64 else 128
    ng = ue // EB
    nu0 = T0 // (NT * ue)
    nu1 = T1 // (NT * ue)

    @functools.partial(
        pl.kernel,
        out_type=jax.ShapeDtypeStruct((NC, N0P, W), jnp.float32),
        mesh=mesh,
        compiler_params=pltpu.CompilerParams(use_tc_tiling_on_sc=False,
                                             needs_layout_passes=False),
        scratch_types=[
            pltpu.VMEM_SHARED((N0P, W), jnp.float32),
            pltpu.VMEM((4, ue), jnp.int32),
            pltpu.VMEM((4, ue), jnp.int32),
            pltpu.VMEM((4, ue), jnp.float32),
            pltpu.VMEM((2, ue, W), jnp.float32),
            pltpu.VMEM((8, W), jnp.float32),
            pltpu.SemaphoreType.DMA,
            pltpu.SemaphoreType.DMA,
            pltpu.SemaphoreType.DMA,
            pltpu.SemaphoreType.DMA,
            pltpu.SemaphoreType.DMA,
            pltpu.SemaphoreType.DMA,
        ],
    )
    def spmm0(h0_hbm, h1_hbm, e0d, e0s, e0v, e1d, e1s, e1v,
              p_hbm, acc, ibd, ibs, ibv, rbuf, zb,
              is0, is1, gs0, gs1, ss0, ss1):
        c = lax.axis_index("c")
        s = lax.axis_index("s")
        base = s * stripe
        isem = [is0, is1]
        gsem = [gs0, gs1]
        ssem = [ss0, ss1]

        z = jnp.zeros((16,), jnp.float32)
        for r in range(8):
            for i in range(W // 16):
                zb[r, pl.ds(16 * i, 16)] = z

        def zrow(i, carry):
            pltpu.sync_copy(zb, acc.at[pl.ds(base + i * 8, 8)])
            return carry
        lax.fori_loop(0, stripe // 8, zrow, 0)
        plsc.subcore_barrier()

        def process(d1, s1, v1, src_hbm, nu):
            def off_u(u):
                return pl.multiple_of((s * nu + u) * ue, 8)

            def idx_issue(u, sp):
                o = off_u(u)
                q = u % 4
                pltpu.async_copy(d1.at[pl.ds(o, ue)], ibd.at[q], isem[sp])
                pltpu.async_copy(s1.at[pl.ds(o, ue)], ibs.at[q], isem[sp])
                pltpu.async_copy(v1.at[pl.ds(o, ue)], ibv.at[q], isem[sp])

            def idx_wait(u, sp):
                o = off_u(u)
                q = u % 4
                pltpu.make_async_copy(d1.at[pl.ds(o, ue)], ibd.at[q], isem[sp]).wait()
                pltpu.make_async_copy(s1.at[pl.ds(o, ue)], ibs.at[q], isem[sp]).wait()
                pltpu.make_async_copy(v1.at[pl.ds(o, ue)], ibv.at[q], isem[sp]).wait()

            def g_issue(u, p):
                q = u % 4
                for h in range(ng):
                    pltpu.async_copy(src_hbm.at[ibs.at[q, pl.ds(h * EB, EB)]],
                                     rbuf.at[p, pl.ds(h * EB, EB)], gsem[p])

            def g_wait(u, p):
                q = u % 4
                for h in range(ng):
                    pltpu.make_async_copy(src_hbm.at[ibs.at[q, pl.ds(h * EB, EB)]],
                                          rbuf.at[p, pl.ds(h * EB, EB)], gsem[p]).wait()

            def s_issue(u, p):
                q = u % 4
                for h in range(ng):
                    pltpu.async_copy(rbuf.at[p, pl.ds(h * EB, EB)],
                                     acc.at[ibd.at[q, pl.ds(h * EB, EB)]], ssem[p], add=True)

            def s_wait(u, p):
                q = u % 4
                for h in range(ng):
                    pltpu.make_async_copy(rbuf.at[p, pl.ds(h * EB, EB)],
                                          acc.at[ibd.at[q, pl.ds(h * EB, EB)]], ssem[p]).wait()

            idx_issue(0, 0)
            idx_wait(0, 0)
            g_issue(0, 0)
            idx_issue(1, 1)

            def body(u, carry):
                p = u % 2
                pn = (u + 1) % 2

                @pl.when(u + 1 < nu)
                def _():
                    @pl.when(u >= 1)
                    def _():
                        _sel(2, pn, lambda i: s_wait(u - 1, i))
                    _sel(2, pn, lambda i: idx_wait(u + 1, i))
                    _sel(2, pn, lambda i: g_issue(u + 1, i))

                @pl.when(u + 2 < nu)
                def _():
                    _sel(2, p, lambda i: idx_issue(u + 2, i))

                _sel(2, p, lambda i: g_wait(u, i))
                _scale_unit(rbuf, p, ibv.at[u % 4], 0, W, ue)
                _sel(2, p, lambda i: s_issue(u, i))
                return carry

            lax.fori_loop(0, nu, body, 0)
            s_wait(nu - 1, (nu - 1) % 2)
            s_wait(nu - 2, (nu - 2) % 2)

        @pl.when(c == 0)
        def _():
            process(e0d, e0s, e0v, h0_hbm, nu0)

        @pl.when(c == 1)
        def _():
            process(e1d, e1s, e1v, h1_hbm, nu1)

        plsc.subcore_barrier()
        pltpu.sync_copy(acc.at[pl.ds(base, stripe)], p_hbm.at[c, pl.ds(base, stripe)])

    return spmm0


@functools.cache
def _make_spmm1(W, R, nblk):
    mesh = plsc.VectorSubcoreMesh(core_axis_name="c", subcore_axis_name="s")
    ue = 256 if W == 64 else 128
    ng = ue // EB
    CAP = SB + ue + 16
    NCH = -(-N1 // R)
    stripe = R // NT

    @functools.partial(
        pl.kernel,
        out_type=jax.ShapeDtypeStruct((NCH * R, W), jnp.float32),
        mesh=mesh,
        compiler_params=pltpu.CompilerParams(use_tc_tiling_on_sc=False,
                                             needs_layout_passes=False),
        scratch_types=[
            pltpu.VMEM_SHARED((R, W), jnp.float32),
            pltpu.VMEM((2, SB), jnp.int32),
            pltpu.VMEM((2, SB), jnp.int32),
            pltpu.VMEM((2, SB), jnp.float32),
            pltpu.VMEM((CAP,), jnp.int32),
            pltpu.VMEM((CAP,), jnp.int32),
            pltpu.VMEM((CAP,), jnp.float32),
            pltpu.VMEM((2, ue), jnp.int32),
            pltpu.VMEM((2, ue, W), jnp.float32),
            pltpu.VMEM((8, W), jnp.float32),
            pltpu.SemaphoreType.DMA,
            pltpu.SemaphoreType.DMA,
            pltpu.SemaphoreType.DMA,
            pltpu.SemaphoreType.DMA,
            pltpu.SemaphoreType.DMA,
            pltpu.SemaphoreType.DMA,
        ],
    )
    def spmm1(src_hbm, d1, s1, v1,
              out_hbm, acc, ed, es, ev, cd, cs, cv, rdst, rbuf, zb,
              is0, is1, gs0, gs1, ss0, ss1):
        c = lax.axis_index("c")
        s = lax.axis_index("s")
        isem = [is0, is1]
        gsem = [gs0, gs1]
        ssem = [ss0, ss1]

        z = jnp.zeros((16,), jnp.float32)
        for r in range(8):
            for i in range(W // 16):
                zb[r, pl.ds(16 * i, 16)] = z

        tile_e0 = s * nblk * SB

        def blk_issue(b, pb):
            boff = pl.multiple_of(tile_e0 + b * SB, 8)
            pltpu.async_copy(d1.at[pl.ds(boff, SB)], ed.at[pb], isem[pb])
            pltpu.async_copy(s1.at[pl.ds(boff, SB)], es.at[pb], isem[pb])
            pltpu.async_copy(v1.at[pl.ds(boff, SB)], ev.at[pb], isem[pb])

        def blk_wait(b, pb):
            boff = pl.multiple_of(tile_e0 + b * SB, 8)
            pltpu.make_async_copy(d1.at[pl.ds(boff, SB)], ed.at[pb], isem[pb]).wait()
            pltpu.make_async_copy(s1.at[pl.ds(boff, SB)], es.at[pb], isem[pb]).wait()
            pltpu.make_async_copy(v1.at[pl.ds(boff, SB)], ev.at[pb], isem[pb]).wait()

        def s_wait(slot):
            for h in range(ng):
                pltpu.make_async_copy(rbuf.at[slot, pl.ds(h * EB, EB)],
                                      acc.at[rdst.at[slot, pl.ds(h * EB, EB)]],
                                      ssem[slot]).wait()

        def scan_and_process(lo):
            blk_issue(0, 0)

            def blk_body(b, ug):
                pb = b % 2

                @pl.when(b + 1 < nblk)
                def _():
                    _sel(2, (b + 1) % 2, lambda i: blk_issue(b + 1, i))

                _sel(2, pb, lambda i: blk_wait(b, i))

                def grp(g, fill):
                    d16 = ed[pb, pl.ds(16 * g, 16)]
                    loc = d16 - lo
                    m = (loc >= 0) & (loc < R)
                    cnt = plsc.all_reduce_population_count(m)[0]
                    plsc.store_compressed(cd.at[pl.ds(fill, 16)], loc, mask=m)
                    plsc.store_compressed(cs.at[pl.ds(fill, 16)], es[pb, pl.ds(16 * g, 16)], mask=m)
                    plsc.store_compressed(cv.at[pl.ds(fill, 16)], ev[pb, pl.ds(16 * g, 16)], mask=m)
                    return fill + cnt

                fill = lax.fori_loop(0, SB // 16, grp, jnp.int32(0))

                zi = jnp.zeros((16,), jnp.int32)
                zf = jnp.zeros((16,), jnp.float32)
                for k in range(ue // 16):
                    cd[pl.ds(fill + 16 * k, 16)] = zi
                    cs[pl.ds(fill + 16 * k, 16)] = zi
                    cv[pl.ds(fill + 16 * k, 16)] = zf

                def unit(k, ug2):
                    slot = ug2 % 2

                    @pl.when(ug2 >= 2)
                    def _():
                        _sel(2, slot, s_wait)

                    koff = k * ue
                    for i in range(ue // 16):
                        rdst[slot, pl.ds(16 * i, 16)] = cd[pl.ds(koff + 16 * i, 16)]

                    def gather(sl):
                        for h in range(ng):
                            pltpu.async_copy(src_hbm.at[cs.at[pl.ds(koff + h * EB, EB)]],
                                             rbuf.at[sl, pl.ds(h * EB, EB)], gsem[sl])
                        for h in range(ng):
                            pltpu.make_async_copy(src_hbm.at[cs.at[pl.ds(koff + h * EB, EB)]],
                                                  rbuf.at[sl, pl.ds(h * EB, EB)], gsem[sl]).wait()

                    _sel(2, slot, gather)
                    _scale_unit(rbuf, slot, cv, koff, W, ue)

                    def scat(sl):
                        for h in range(ng):
                            pltpu.async_copy(rbuf.at[sl, pl.ds(h * EB, EB)],
                                             acc.at[rdst.at[sl, pl.ds(h * EB, EB)]],
                                             ssem[sl], add=True)

                    _sel(2, slot, scat)
                    return ug2 + 1

                return lax.fori_loop(0, (fill + ue - 1) // ue, unit, ug)

            ug = lax.fori_loop(0, nblk, blk_body, jnp.int32(0))
            for dd in range(2):
                @pl.when(ug >= dd + 1)
                def _():
                    _sel(2, (ug - 1 - dd) % 2, s_wait)

        def chunk_body(ch, carry):
            chunk = c * (NCH // NC) + ch
            base = pl.multiple_of(chunk * R + s * stripe, 8)

            def zrow(i, carry2):
                pltpu.sync_copy(zb, acc.at[pl.ds(s * stripe + i * 8, 8)])
                return carry2
            lax.fori_loop(0, stripe // 8, zrow, 0)
            plsc.subcore_barrier()
            scan_and_process(chunk * R)
            plsc.subcore_barrier()
            pltpu.sync_copy(acc.at[pl.ds(s * stripe, stripe)], out_hbm.at[pl.ds(base, stripe)])
            plsc.subcore_barrier()
            return carry

        lax.fori_loop(0, NCH // NC, chunk_body, 0)

    return spmm1


def _pad_zero(a, total):
    return jnp.concatenate([a, jnp.zeros((total - a.shape[0],), a.dtype)])


def _pad_oob(d, s, v, total):
    pad = total - d.shape[0]
    return (jnp.concatenate([d, jnp.full((pad,), jnp.int32(1 << 29))]),
            jnp.concatenate([s, jnp.zeros((pad,), s.dtype)]),
            jnp.concatenate([v, jnp.zeros((pad,), v.dtype)]))


def kernel(X0, X1, B1_row, B1_col, B1_val, L0_row, L0_col, L0_val, L1_row, L1_col, L1_val, W1_0L, W1_0B, W1_0I, W1_1L, W1_1B, W1_1I, W2_0L, W2_0B, W2_0I, W2_1L, W2_1B, W2_1I, W3_0L, W3_0B, W3_0I, W3_1L, W3_1B, W3_1I, Wfc):
    s0 = (_pad_zero(L0_row, T0), _pad_zero(L0_col, T0), _pad_zero(L0_val, T0))
    s1 = (_pad_zero(B1_row, T1), _pad_zero(B1_col, T1), _pad_zero(B1_val, T1))
    uL = _pad_oob(L1_row, L1_col, L1_val, NBL1 * NT * SB)
    uB = _pad_oob(B1_col, B1_row, B1_val, NBB1 * NT * SB)

    Ws = [(W1_0L, W1_0B, W1_0I, W1_1L, W1_1B, W1_1I),
          (W2_0L, W2_0B, W2_0I, W2_1L, W2_1B, W2_1I),
          (W3_0L, W3_0B, W3_0I, W3_1L, W3_1B, W3_1I)]

    h0 = jnp.concatenate([X0, jnp.zeros((N0P - N0, X0.shape[1]), jnp.float32)])
    h1 = X1
    for li, (WL0, WB0, WI0, WL1, WB1, WI1) in enumerate(Ws):
        W = h0.shape[1]
        R = 8192 if W == 128 else 16384
        spmm0 = _make_spmm0(W)
        spmm1L = _make_spmm1(W, R, NBL1)
        spmm1B = _make_spmm1(W, R, NBB1)
        P = spmm0(h0, h1, *s0, *s1)
        SL = spmm1L(h1, *uL)
        SBt = spmm1B(h0, *uB)
        h0 = _combine(P, None, h0, WL0, WB0, WI0, pair=True, blk=2048)
        h1 = _combine(SL, SBt, h1, WL1, WB1, WI1, pair=False, blk=2000)

    out0 = _fc(h0, Wfc, blk=2048)[:N0]
    out1 = _fc(h1, Wfc, blk=2000)
    return (out0, out1)


# speedup vs baseline: 1.0059x; 1.0022x over previous
import functools

import jax
import jax.numpy as jnp
from jax import lax
from jax.experimental import pallas as pl
from jax.experimental.pallas import tpu as pltpu
from jax.experimental.pallas import tpu_sc as plsc

N0 = 10000
N0P = 10240
N1 = 160000
EB = 128
SB = 2048
NT = 16
NC = 2

T0 = 335872
T1 = 327680
NBL1 = -(-800000 // (NT * SB))
NBB1 = -(-320000 // (NT * SB))


def _comb_body(pair, x1_ref, x2_ref, x3_ref, w1_ref, w2_ref, w3_ref, o_ref):
    x1 = x1_ref[...]
    if pair:
        x1, x2 = x1[0], x1[1]
    else:
        x2 = x2_ref[...]
    m = (jnp.dot(x1, w1_ref[...], preferred_element_type=jnp.float32)
         + jnp.dot(x2, w2_ref[...], preferred_element_type=jnp.float32)
         + jnp.dot(x3_ref[...], w3_ref[...], preferred_element_type=jnp.float32))
    o_ref[...] = jnp.tanh(m)


def _combine(x1, x2, x3, w1, w2, w3, pair, blk):
    n, d = x3.shape
    if pair:
        s1 = pl.BlockSpec((2, blk, d), lambda i: (0, i, 0))
        s2 = pl.BlockSpec((1, 1), lambda i: (0, 0))
        x2 = jnp.zeros((1, 1), jnp.float32)
    else:
        s1 = pl.BlockSpec((blk, d), lambda i: (i, 0))
        s2 = pl.BlockSpec((blk, d), lambda i: (i, 0))
    dout = w1.shape[1]
    wspec = pl.BlockSpec((d, dout), lambda i: (0, 0))
    return pl.pallas_call(
        functools.partial(_comb_body, pair),
        grid=(n // blk,),
        in_specs=[s1, s2, pl.BlockSpec((blk, d), lambda i: (i, 0)),
                  wspec, wspec, wspec],
        out_specs=pl.BlockSpec((blk, dout), lambda i: (i, 0)),
        out_shape=jax.ShapeDtypeStruct((n, dout), jnp.float32),
    )(x1, x2, x3, w1, w2, w3)


def _fc_body(x_ref, w_ref, o_ref):
    o_ref[...] = jnp.dot(x_ref[...], w_ref[...], preferred_element_type=jnp.float32)


def _fc(x, w, blk):
    n, d = x.shape
    dout = w.shape[1]
    return pl.pallas_call(
        _fc_body,
        grid=(n // blk,),
        in_specs=[pl.BlockSpec((blk, d), lambda i: (i, 0)),
                  pl.BlockSpec((d, dout), lambda i: (0, 0))],
        out_specs=pl.BlockSpec((blk, dout), lambda i: (i, 0)),
        out_shape=jax.ShapeDtypeStruct((n, dout), jnp.float32),
    )(x, w)


def _sel(n, traced_idx, f):
    for i in range(n):
        @pl.when(traced_idx == i)
        def _(i=i):
            f(i)


def _scale_unit(rbuf, p, val_ref, voff, W, ue):
    def grp(g, carry):
        vv = val_ref[pl.ds(voff + 16 * g, 16)]
        for e in range(16):
            splat = jnp.broadcast_to(vv[e], (16,))
            r = 16 * g + e
            for f in range(W // 16):
                rbuf[p, r, pl.ds(16 * f, 16)] = rbuf[p, r, pl.ds(16 * f, 16)] * splat
        return carry

    lax.fori_loop(0, ue // 16, grp, 0)


@functools.cache
def _make_spmm0(W):
    mesh = plsc.VectorSubcoreMesh(core_axis_name="c", subcore_axis_name="s")
    stripe = N0P // NT
    ue = 256 if W == 64 else 128
    ng = ue // EB
    nu0 = T0 // (NT * ue)
    nu1 = T1 // (NT * ue)

    @functools.partial(
        pl.kernel,
        out_type=jax.ShapeDtypeStruct((NC, N0P, W), jnp.float32),
        mesh=mesh,
        compiler_params=pltpu.CompilerParams(use_tc_tiling_on_sc=False,
                                             needs_layout_passes=False),
        scratch_types=[
            pltpu.VMEM_SHARED((N0P, W), jnp.float32),
            pltpu.VMEM((4, ue), jnp.int32),
            pltpu.VMEM((4, ue), jnp.int32),
            pltpu.VMEM((4, ue), jnp.float32),
            pltpu.VMEM((2, ue, W), jnp.float32),
            pltpu.VMEM((8, W), jnp.float32),
            pltpu.SemaphoreType.DMA,
            pltpu.SemaphoreType.DMA,
            pltpu.SemaphoreType.DMA,
            pltpu.SemaphoreType.DMA,
            pltpu.SemaphoreType.DMA,
            pltpu.SemaphoreType.DMA,
        ],
    )
    def spmm0(h0_hbm, h1_hbm, e0d, e0s, e0v, e1d, e1s, e1v,
              p_hbm, acc, ibd, ibs, ibv, rbuf, zb,
              is0, is1, gs0, gs1, ss0, ss1):
        c = lax.axis_index("c")
        s = lax.axis_index("s")
        base = s * stripe
        isem = [is0, is1]
        gsem = [gs0, gs1]
        ssem = [ss0, ss1]

        z = jnp.zeros((16,), jnp.float32)
        for r in range(8):
            for i in range(W // 16):
                zb[r, pl.ds(16 * i, 16)] = z

        def zrow(i, carry):
            pltpu.sync_copy(zb, acc.at[pl.ds(base + i * 8, 8)])
            return carry
        lax.fori_loop(0, stripe // 8, zrow, 0)
        plsc.subcore_barrier()

        def process(d1, s1, v1, src_hbm, nu):
            def off_u(u):
                return pl.multiple_of((s * nu + u) * ue, 8)

            def idx_issue(u, sp):
                o = off_u(u)
                q = u % 4
                pltpu.async_copy(d1.at[pl.ds(o, ue)], ibd.at[q], isem[sp])
                pltpu.async_copy(s1.at[pl.ds(o, ue)], ibs.at[q], isem[sp])
                pltpu.async_copy(v1.at[pl.ds(o, ue)], ibv.at[q], isem[sp])

            def idx_wait(u, sp):
                o = off_u(u)
                q = u % 4
                pltpu.make_async_copy(d1.at[pl.ds(o, ue)], ibd.at[q], isem[sp]).wait()
                pltpu.make_async_copy(s1.at[pl.ds(o, ue)], ibs.at[q], isem[sp]).wait()
                pltpu.make_async_copy(v1.at[pl.ds(o, ue)], ibv.at[q], isem[sp]).wait()

            def g_issue(u, p):
                q = u % 4
                for h in range(ng):
                    pltpu.async_copy(src_hbm.at[ibs.at[q, pl.ds(h * EB, EB)]],
                                     rbuf.at[p, pl.ds(h * EB, EB)], gsem[p])

            def g_wait(u, p):
                q = u % 4
                for h in range(ng):
                    pltpu.make_async_copy(src_hbm.at[ibs.at[q, pl.ds(h * EB, EB)]],
                                          rbuf.at[p, pl.ds(h * EB, EB)], gsem[p]).wait()

            def s_issue(u, p):
                q = u % 4
                for h in range(ng):
                    pltpu.async_copy(rbuf.at[p, pl.ds(h * EB, EB)],
                                     acc.at[ibd.at[q, pl.ds(h * EB, EB)]], ssem[p], add=True)

            def s_wait(u, p):
                q = u % 4
                for h in range(ng):
                    pltpu.make_async_copy(rbuf.at[p, pl.ds(h * EB, EB)],
                                          acc.at[ibd.at[q, pl.ds(h * EB, EB)]], ssem[p]).wait()

            idx_issue(0, 0)
            idx_wait(0, 0)
            g_issue(0, 0)
            idx_issue(1, 1)

            def body(u, carry):
                p = u % 2
                pn = (u + 1) % 2

                @pl.when(u + 1 < nu)
                def _():
                    @pl.when(u >= 1)
                    def _():
                        _sel(2, pn, lambda i: s_wait(u - 1, i))
                    _sel(2, pn, lambda i: idx_wait(u + 1, i))
                    _sel(2, pn, lambda i: g_issue(u + 1, i))

                @pl.when(u + 2 < nu)
                def _():
                    _sel(2, p, lambda i: idx_issue(u + 2, i))

                _sel(2, p, lambda i: g_wait(u, i))
                _scale_unit(rbuf, p, ibv.at[u % 4], 0, W, ue)
                _sel(2, p, lambda i: s_issue(u, i))
                return carry

            lax.fori_loop(0, nu, body, 0)
            s_wait(nu - 1, (nu - 1) % 2)
            s_wait(nu - 2, (nu - 2) % 2)

        @pl.when(c == 0)
        def _():
            process(e0d, e0s, e0v, h0_hbm, nu0)

        @pl.when(c == 1)
        def _():
            process(e1d, e1s, e1v, h1_hbm, nu1)

        plsc.subcore_barrier()
        pltpu.sync_copy(acc.at[pl.ds(base, stripe)], p_hbm.at[c, pl.ds(base, stripe)])

    return spmm0


@functools.cache
def _make_spmm1(W, R, nblk):
    mesh = plsc.VectorSubcoreMesh(core_axis_name="c", subcore_axis_name="s")
    ue = 256 if W == 64 else 128
    ng = ue // EB
    CAP = SB + ue + 16
    NCH = -(-N1 // R)
    stripe = R // NT

    @functools.partial(
        pl.kernel,
        out_type=jax.ShapeDtypeStruct((NCH * R, W), jnp.float32),
        mesh=mesh,
        compiler_params=pltpu.CompilerParams(use_tc_tiling_on_sc=False,
                                             needs_layout_passes=False),
        scratch_types=[
            pltpu.VMEM_SHARED((R, W), jnp.float32),
            pltpu.VMEM((2, SB), jnp.int32),
            pltpu.VMEM((2, SB), jnp.int32),
            pltpu.VMEM((2, SB), jnp.float32),
            pltpu.VMEM((CAP,), jnp.int32),
            pltpu.VMEM((CAP,), jnp.int32),
            pltpu.VMEM((CAP,), jnp.float32),
            pltpu.VMEM((2, ue), jnp.int32),
            pltpu.VMEM((2, ue, W), jnp.float32),
            pltpu.VMEM((8, W), jnp.float32),
            pltpu.SemaphoreType.DMA,
            pltpu.SemaphoreType.DMA,
            pltpu.SemaphoreType.DMA,
            pltpu.SemaphoreType.DMA,
            pltpu.SemaphoreType.DMA,
            pltpu.SemaphoreType.DMA,
        ],
    )
    def spmm1(src_hbm, d1, s1, v1,
              out_hbm, acc, ed, es, ev, cd, cs, cv, rdst, rbuf, zb,
              is0, is1, gs0, gs1, ss0, ss1):
        c = lax.axis_index("c")
        s = lax.axis_index("s")
        isem = [is0, is1]
        gsem = [gs0, gs1]
        ssem = [ss0, ss1]

        z = jnp.zeros((16,), jnp.float32)
        for r in range(8):
            for i in range(W // 16):
                zb[r, pl.ds(16 * i, 16)] = z

        tile_e0 = s * nblk * SB

        def blk_issue(b, pb):
            boff = pl.multiple_of(tile_e0 + b * SB, 8)
            pltpu.async_copy(d1.at[pl.ds(boff, SB)], ed.at[pb], isem[pb])
            pltpu.async_copy(s1.at[pl.ds(boff, SB)], es.at[pb], isem[pb])
            pltpu.async_copy(v1.at[pl.ds(boff, SB)], ev.at[pb], isem[pb])

        def blk_wait(b, pb):
            boff = pl.multiple_of(tile_e0 + b * SB, 8)
            pltpu.make_async_copy(d1.at[pl.ds(boff, SB)], ed.at[pb], isem[pb]).wait()
            pltpu.make_async_copy(s1.at[pl.ds(boff, SB)], es.at[pb], isem[pb]).wait()
            pltpu.make_async_copy(v1.at[pl.ds(boff, SB)], ev.at[pb], isem[pb]).wait()

        def s_wait(slot):
            for h in range(ng):
                pltpu.make_async_copy(rbuf.at[slot, pl.ds(h * EB, EB)],
                                      acc.at[rdst.at[slot, pl.ds(h * EB, EB)]],
                                      ssem[slot]).wait()

        def scan_and_process(lo):
            blk_issue(0, 0)

            def blk_body(b, ug):
                pb = b % 2

                @pl.when(b + 1 < nblk)
                def _():
                    _sel(2, (b + 1) % 2, lambda i: blk_issue(b + 1, i))

                _sel(2, pb, lambda i: blk_wait(b, i))

                def grp(g8, fill):
                    locs, ms, cnts = [], [], []
                    for j in range(8):
                        d16 = ed[pb, pl.ds(128 * g8 + 16 * j, 16)]
                        loc = d16 - lo
                        m = (loc >= 0) & (loc < R)
                        locs.append(loc)
                        ms.append(m)
                        cnts.append(plsc.all_reduce_population_count(m)[0])
                    for j in range(8):
                        plsc.store_compressed(cd.at[pl.ds(fill, 16)], locs[j], mask=ms[j])
                        plsc.store_compressed(cs.at[pl.ds(fill, 16)],
                                              es[pb, pl.ds(128 * g8 + 16 * j, 16)], mask=ms[j])
                        plsc.store_compressed(cv.at[pl.ds(fill, 16)],
                                              ev[pb, pl.ds(128 * g8 + 16 * j, 16)], mask=ms[j])
                        fill = fill + cnts[j]
                    return fill

                fill = lax.fori_loop(0, SB // 128, grp, jnp.int32(0))

                zi = jnp.zeros((16,), jnp.int32)
                zf = jnp.zeros((16,), jnp.float32)
                for k in range(ue // 16):
                    cd[pl.ds(fill + 16 * k, 16)] = zi
                    cs[pl.ds(fill + 16 * k, 16)] = zi
                    cv[pl.ds(fill + 16 * k, 16)] = zf

                def unit(k, ug2):
                    slot = ug2 % 2

                    @pl.when(ug2 >= 2)
                    def _():
                        _sel(2, slot, s_wait)

                    koff = k * ue
                    for i in range(ue // 16):
                        rdst[slot, pl.ds(16 * i, 16)] = cd[pl.ds(koff + 16 * i, 16)]

                    def gather(sl):
                        for h in range(ng):
                            pltpu.async_copy(src_hbm.at[cs.at[pl.ds(koff + h * EB, EB)]],
                                             rbuf.at[sl, pl.ds(h * EB, EB)], gsem[sl])
                        for h in range(ng):
                            pltpu.make_async_copy(src_hbm.at[cs.at[pl.ds(koff + h * EB, EB)]],
                                                  rbuf.at[sl, pl.ds(h * EB, EB)], gsem[sl]).wait()

                    _sel(2, slot, gather)
                    _scale_unit(rbuf, slot, cv, koff, W, ue)

                    def scat(sl):
                        for h in range(ng):
                            pltpu.async_copy(rbuf.at[sl, pl.ds(h * EB, EB)],
                                             acc.at[rdst.at[sl, pl.ds(h * EB, EB)]],
                                             ssem[sl], add=True)

                    _sel(2, slot, scat)
                    return ug2 + 1

                return lax.fori_loop(0, (fill + ue - 1) // ue, unit, ug)

            ug = lax.fori_loop(0, nblk, blk_body, jnp.int32(0))
            for dd in range(2):
                @pl.when(ug >= dd + 1)
                def _():
                    _sel(2, (ug - 1 - dd) % 2, s_wait)

        def chunk_body(ch, carry):
            chunk = c * (NCH // NC) + ch
            base = pl.multiple_of(chunk * R + s * stripe, 8)

            def zrow(i, carry2):
                pltpu.sync_copy(zb, acc.at[pl.ds(s * stripe + i * 8, 8)])
                return carry2
            lax.fori_loop(0, stripe // 8, zrow, 0)
            plsc.subcore_barrier()
            scan_and_process(chunk * R)
            plsc.subcore_barrier()
            pltpu.sync_copy(acc.at[pl.ds(s * stripe, stripe)], out_hbm.at[pl.ds(base, stripe)])
            plsc.subcore_barrier()
            return carry

        lax.fori_loop(0, NCH // NC, chunk_body, 0)

    return spmm1


def _pad_zero(a, total):
    return jnp.concatenate([a, jnp.zeros((total - a.shape[0],), a.dtype)])


def _pad_oob(d, s, v, total):
    pad = total - d.shape[0]
    return (jnp.concatenate([d, jnp.full((pad,), jnp.int32(1 << 29))]),
            jnp.concatenate([s, jnp.zeros((pad,), s.dtype)]),
            jnp.concatenate([v, jnp.zeros((pad,), v.dtype)]))


def kernel(X0, X1, B1_row, B1_col, B1_val, L0_row, L0_col, L0_val, L1_row, L1_col, L1_val, W1_0L, W1_0B, W1_0I, W1_1L, W1_1B, W1_1I, W2_0L, W2_0B, W2_0I, W2_1L, W2_1B, W2_1I, W3_0L, W3_0B, W3_0I, W3_1L, W3_1B, W3_1I, Wfc):
    s0 = (_pad_zero(L0_row, T0), _pad_zero(L0_col, T0), _pad_zero(L0_val, T0))
    s1 = (_pad_zero(B1_row, T1), _pad_zero(B1_col, T1), _pad_zero(B1_val, T1))
    uL = _pad_oob(L1_row, L1_col, L1_val, NBL1 * NT * SB)
    uB = _pad_oob(B1_col, B1_row, B1_val, NBB1 * NT * SB)

    Ws = [(W1_0L, W1_0B, W1_0I, W1_1L, W1_1B, W1_1I),
          (W2_0L, W2_0B, W2_0I, W2_1L, W2_1B, W2_1I),
          (W3_0L, W3_0B, W3_0I, W3_1L, W3_1B, W3_1I)]

    h0 = jnp.concatenate([X0, jnp.zeros((N0P - N0, X0.shape[1]), jnp.float32)])
    h1 = X1
    for li, (WL0, WB0, WI0, WL1, WB1, WI1) in enumerate(Ws):
        W = h0.shape[1]
        R = 8192 if W == 128 else 16384
        spmm0 = _make_spmm0(W)
        spmm1L = _make_spmm1(W, R, NBL1)
        spmm1B = _make_spmm1(W, R, NBB1)
        P = spmm0(h0, h1, *s0, *s1)
        SL = spmm1L(h1, *uL)
        SBt = spmm1B(h0, *uB)
        h0 = _combine(P, None, h0, WL0, WB0, WI0, pair=True, blk=2048)
        h1 = _combine(SL, SBt, h1, WL1, WB1, WI1, pair=False, blk=2000)

    out0 = _fc(h0, Wfc, blk=2048)[:N0]
    out1 = _fc(h1, Wfc, blk=2000)
    return (out0, out1)


# speedup vs baseline: 1.0116x; 1.0056x over previous
import functools

import jax
import jax.numpy as jnp
from jax import lax
from jax.experimental import pallas as pl
from jax.experimental.pallas import tpu as pltpu
from jax.experimental.pallas import tpu_sc as plsc

N0 = 10000
N0P = 10240
N1 = 160000
EB = 128
SB = 2048
NT = 16
NC = 2

T0 = 335872
T1 = 327680
NBL1 = -(-800000 // (NT * SB))
NBB1 = -(-320000 // (NT * SB))


def _comb_body(pair, x1_ref, x2_ref, x3_ref, w1_ref, w2_ref, w3_ref, o_ref):
    x1 = x1_ref[...]
    if pair:
        x1, x2 = x1[0], x1[1]
    else:
        x2 = x2_ref[...]
    m = (jnp.dot(x1, w1_ref[...], preferred_element_type=jnp.float32)
         + jnp.dot(x2, w2_ref[...], preferred_element_type=jnp.float32)
         + jnp.dot(x3_ref[...], w3_ref[...], preferred_element_type=jnp.float32))
    o_ref[...] = jnp.tanh(m)


def _combine(x1, x2, x3, w1, w2, w3, pair, blk):
    n, d = x3.shape
    if pair:
        s1 = pl.BlockSpec((2, blk, d), lambda i: (0, i, 0))
        s2 = pl.BlockSpec((1, 1), lambda i: (0, 0))
        x2 = jnp.zeros((1, 1), jnp.float32)
    else:
        s1 = pl.BlockSpec((blk, d), lambda i: (i, 0))
        s2 = pl.BlockSpec((blk, d), lambda i: (i, 0))
    dout = w1.shape[1]
    wspec = pl.BlockSpec((d, dout), lambda i: (0, 0))
    return pl.pallas_call(
        functools.partial(_comb_body, pair),
        grid=(n // blk,),
        in_specs=[s1, s2, pl.BlockSpec((blk, d), lambda i: (i, 0)),
                  wspec, wspec, wspec],
        out_specs=pl.BlockSpec((blk, dout), lambda i: (i, 0)),
        out_shape=jax.ShapeDtypeStruct((n, dout), jnp.float32),
    )(x1, x2, x3, w1, w2, w3)


def _fc_body(x_ref, w_ref, o_ref):
    o_ref[...] = jnp.dot(x_ref[...], w_ref[...], preferred_element_type=jnp.float32)


def _fc(x, w, blk):
    n, d = x.shape
    dout = w.shape[1]
    return pl.pallas_call(
        _fc_body,
        grid=(n // blk,),
        in_specs=[pl.BlockSpec((blk, d), lambda i: (i, 0)),
                  pl.BlockSpec((d, dout), lambda i: (0, 0))],
        out_specs=pl.BlockSpec((blk, dout), lambda i: (i, 0)),
        out_shape=jax.ShapeDtypeStruct((n, dout), jnp.float32),
    )(x, w)


def _sel(n, traced_idx, f):
    for i in range(n):
        @pl.when(traced_idx == i)
        def _(i=i):
            f(i)


def _scale_unit(rbuf, p, val_ref, voff, W, ue):
    def grp(g, carry):
        vv = val_ref[pl.ds(voff + 16 * g, 16)]
        for e in range(16):
            splat = jnp.broadcast_to(vv[e], (16,))
            r = 16 * g + e
            for f in range(W // 16):
                rbuf[p, r, pl.ds(16 * f, 16)] = rbuf[p, r, pl.ds(16 * f, 16)] * splat
        return carry

    lax.fori_loop(0, ue // 16, grp, 0)


@functools.cache
def _make_spmm0(W):
    mesh = plsc.VectorSubcoreMesh(core_axis_name="c", subcore_axis_name="s")
    stripe = N0P // NT
    ue = 256 if W == 64 else 128
    ng = ue // EB
    nu0 = T0 // (NT * ue)
    nu1 = T1 // (NT * ue)

    @functools.partial(
        pl.kernel,
        out_type=jax.ShapeDtypeStruct((NC, N0P, W), jnp.float32),
        mesh=mesh,
        compiler_params=pltpu.CompilerParams(use_tc_tiling_on_sc=False,
                                             needs_layout_passes=False),
        scratch_types=[
            pltpu.VMEM_SHARED((N0P, W), jnp.float32),
            pltpu.VMEM((4, ue), jnp.int32),
            pltpu.VMEM((4, ue), jnp.int32),
            pltpu.VMEM((4, ue), jnp.float32),
            pltpu.VMEM((2, ue, W), jnp.float32),
            pltpu.VMEM((64, W), jnp.float32),
            pltpu.SemaphoreType.DMA,
            pltpu.SemaphoreType.DMA,
            pltpu.SemaphoreType.DMA,
            pltpu.SemaphoreType.DMA,
            pltpu.SemaphoreType.DMA,
            pltpu.SemaphoreType.DMA,
        ],
    )
    def spmm0(h0_hbm, h1_hbm, e0d, e0s, e0v, e1d, e1s, e1v,
              p_hbm, acc, ibd, ibs, ibv, rbuf, zb,
              is0, is1, gs0, gs1, ss0, ss1):
        c = lax.axis_index("c")
        s = lax.axis_index("s")
        base = s * stripe
        isem = [is0, is1]
        gsem = [gs0, gs1]
        ssem = [ss0, ss1]

        z = jnp.zeros((16,), jnp.float32)

        def zfill(r, carry):
            for i in range(W // 16):
                zb[r, pl.ds(16 * i, 16)] = z
            return carry
        lax.fori_loop(0, 64, zfill, 0)

        def zrow(i, carry):
            pltpu.sync_copy(zb, acc.at[pl.ds(base + i * 64, 64)])
            return carry
        lax.fori_loop(0, stripe // 64, zrow, 0)
        plsc.subcore_barrier()

        def process(d1, s1, v1, src_hbm, nu):
            def off_u(u):
                return pl.multiple_of((s * nu + u) * ue, 8)

            def idx_issue(u, sp):
                o = off_u(u)
                q = u % 4
                pltpu.async_copy(d1.at[pl.ds(o, ue)], ibd.at[q], isem[sp])
                pltpu.async_copy(s1.at[pl.ds(o, ue)], ibs.at[q], isem[sp])
                pltpu.async_copy(v1.at[pl.ds(o, ue)], ibv.at[q], isem[sp])

            def idx_wait(u, sp):
                o = off_u(u)
                q = u % 4
                pltpu.make_async_copy(d1.at[pl.ds(o, ue)], ibd.at[q], isem[sp]).wait()
                pltpu.make_async_copy(s1.at[pl.ds(o, ue)], ibs.at[q], isem[sp]).wait()
                pltpu.make_async_copy(v1.at[pl.ds(o, ue)], ibv.at[q], isem[sp]).wait()

            def g_issue(u, p):
                q = u % 4
                for h in range(ng):
                    pltpu.async_copy(src_hbm.at[ibs.at[q, pl.ds(h * EB, EB)]],
                                     rbuf.at[p, pl.ds(h * EB, EB)], gsem[p])

            def g_wait(u, p):
                q = u % 4
                for h in range(ng):
                    pltpu.make_async_copy(src_hbm.at[ibs.at[q, pl.ds(h * EB, EB)]],
                                          rbuf.at[p, pl.ds(h * EB, EB)], gsem[p]).wait()

            def s_issue(u, p):
                q = u % 4
                for h in range(ng):
                    pltpu.async_copy(rbuf.at[p, pl.ds(h * EB, EB)],
                                     acc.at[ibd.at[q, pl.ds(h * EB, EB)]], ssem[p], add=True)

            def s_wait(u, p):
                q = u % 4
                for h in range(ng):
                    pltpu.make_async_copy(rbuf.at[p, pl.ds(h * EB, EB)],
                                          acc.at[ibd.at[q, pl.ds(h * EB, EB)]], ssem[p]).wait()

            idx_issue(0, 0)
            idx_wait(0, 0)
            g_issue(0, 0)
            idx_issue(1, 1)

            def body(u, carry):
                p = u % 2
                pn = (u + 1) % 2

                @pl.when(u + 1 < nu)
                def _():
                    @pl.when(u >= 1)
                    def _():
                        _sel(2, pn, lambda i: s_wait(u - 1, i))
                    _sel(2, pn, lambda i: idx_wait(u + 1, i))
                    _sel(2, pn, lambda i: g_issue(u + 1, i))

                @pl.when(u + 2 < nu)
                def _():
                    _sel(2, p, lambda i: idx_issue(u + 2, i))

                _sel(2, p, lambda i: g_wait(u, i))
                _scale_unit(rbuf, p, ibv.at[u % 4], 0, W, ue)
                _sel(2, p, lambda i: s_issue(u, i))
                return carry

            lax.fori_loop(0, nu, body, 0)
            s_wait(nu - 1, (nu - 1) % 2)
            s_wait(nu - 2, (nu - 2) % 2)

        @pl.when(c == 0)
        def _():
            process(e0d, e0s, e0v, h0_hbm, nu0)

        @pl.when(c == 1)
        def _():
            process(e1d, e1s, e1v, h1_hbm, nu1)

        plsc.subcore_barrier()
        pltpu.sync_copy(acc.at[pl.ds(base, stripe)], p_hbm.at[c, pl.ds(base, stripe)])

    return spmm0


@functools.cache
def _make_spmm1(W, R, nblk):
    mesh = plsc.VectorSubcoreMesh(core_axis_name="c", subcore_axis_name="s")
    ue = 256 if W == 64 else 128
    ng = ue // EB
    CAP = SB + ue + 16
    NCH = -(-N1 // R)
    stripe = R // NT

    @functools.partial(
        pl.kernel,
        out_type=jax.ShapeDtypeStruct((NCH * R, W), jnp.float32),
        mesh=mesh,
        compiler_params=pltpu.CompilerParams(use_tc_tiling_on_sc=False,
                                             needs_layout_passes=False),
        scratch_types=[
            pltpu.VMEM_SHARED((R, W), jnp.float32),
            pltpu.VMEM((2, SB), jnp.int32),
            pltpu.VMEM((2, SB), jnp.int32),
            pltpu.VMEM((2, SB), jnp.float32),
            pltpu.VMEM((CAP,), jnp.int32),
            pltpu.VMEM((CAP,), jnp.int32),
            pltpu.VMEM((CAP,), jnp.float32),
            pltpu.VMEM((2, ue), jnp.int32),
            pltpu.VMEM((2, ue, W), jnp.float32),
            pltpu.VMEM((64, W), jnp.float32),
            pltpu.SemaphoreType.DMA,
            pltpu.SemaphoreType.DMA,
            pltpu.SemaphoreType.DMA,
            pltpu.SemaphoreType.DMA,
            pltpu.SemaphoreType.DMA,
            pltpu.SemaphoreType.DMA,
        ],
    )
    def spmm1(src_hbm, d1, s1, v1,
              out_hbm, acc, ed, es, ev, cd, cs, cv, rdst, rbuf, zb,
              is0, is1, gs0, gs1, ss0, ss1):
        c = lax.axis_index("c")
        s = lax.axis_index("s")
        isem = [is0, is1]
        gsem = [gs0, gs1]
        ssem = [ss0, ss1]

        z = jnp.zeros((16,), jnp.float32)

        def zfill(r, carry):
            for i in range(W // 16):
                zb[r, pl.ds(16 * i, 16)] = z
            return carry
        lax.fori_loop(0, 64, zfill, 0)

        tile_e0 = s * nblk * SB

        def blk_issue(b, pb):
            boff = pl.multiple_of(tile_e0 + b * SB, 8)
            pltpu.async_copy(d1.at[pl.ds(boff, SB)], ed.at[pb], isem[pb])
            pltpu.async_copy(s1.at[pl.ds(boff, SB)], es.at[pb], isem[pb])
            pltpu.async_copy(v1.at[pl.ds(boff, SB)], ev.at[pb], isem[pb])

        def blk_wait(b, pb):
            boff = pl.multiple_of(tile_e0 + b * SB, 8)
            pltpu.make_async_copy(d1.at[pl.ds(boff, SB)], ed.at[pb], isem[pb]).wait()
            pltpu.make_async_copy(s1.at[pl.ds(boff, SB)], es.at[pb], isem[pb]).wait()
            pltpu.make_async_copy(v1.at[pl.ds(boff, SB)], ev.at[pb], isem[pb]).wait()

        def s_wait(slot):
            for h in range(ng):
                pltpu.make_async_copy(rbuf.at[slot, pl.ds(h * EB, EB)],
                                      acc.at[rdst.at[slot, pl.ds(h * EB, EB)]],
                                      ssem[slot]).wait()

        def scan_and_process(lo):
            blk_issue(0, 0)

            def blk_body(b, ug):
                pb = b % 2

                @pl.when(b + 1 < nblk)
                def _():
                    _sel(2, (b + 1) % 2, lambda i: blk_issue(b + 1, i))

                _sel(2, pb, lambda i: blk_wait(b, i))

                def grp(g8, fill):
                    locs, ms, cnts = [], [], []
                    for j in range(8):
                        d16 = ed[pb, pl.ds(128 * g8 + 16 * j, 16)]
                        loc = d16 - lo
                        m = (loc >= 0) & (loc < R)
                        locs.append(loc)
                        ms.append(m)
                        cnts.append(plsc.all_reduce_population_count(m)[0])
                    for j in range(8):
                        plsc.store_compressed(cd.at[pl.ds(fill, 16)], locs[j], mask=ms[j])
                        plsc.store_compressed(cs.at[pl.ds(fill, 16)],
                                              es[pb, pl.ds(128 * g8 + 16 * j, 16)], mask=ms[j])
                        plsc.store_compressed(cv.at[pl.ds(fill, 16)],
                                              ev[pb, pl.ds(128 * g8 + 16 * j, 16)], mask=ms[j])
                        fill = fill + cnts[j]
                    return fill

                fill = lax.fori_loop(0, SB // 128, grp, jnp.int32(0))

                zi = jnp.zeros((16,), jnp.int32)
                zf = jnp.zeros((16,), jnp.float32)
                for k in range(ue // 16):
                    cd[pl.ds(fill + 16 * k, 16)] = zi
                    cs[pl.ds(fill + 16 * k, 16)] = zi
                    cv[pl.ds(fill + 16 * k, 16)] = zf

                def unit(k, ug2):
                    slot = ug2 % 2

                    @pl.when(ug2 >= 2)
                    def _():
                        _sel(2, slot, s_wait)

                    koff = k * ue
                    for i in range(ue // 16):
                        rdst[slot, pl.ds(16 * i, 16)] = cd[pl.ds(koff + 16 * i, 16)]

                    def gather(sl):
                        for h in range(ng):
                            pltpu.async_copy(src_hbm.at[cs.at[pl.ds(koff + h * EB, EB)]],
                                             rbuf.at[sl, pl.ds(h * EB, EB)], gsem[sl])
                        for h in range(ng):
                            pltpu.make_async_copy(src_hbm.at[cs.at[pl.ds(koff + h * EB, EB)]],
                                                  rbuf.at[sl, pl.ds(h * EB, EB)], gsem[sl]).wait()

                    _sel(2, slot, gather)
                    _scale_unit(rbuf, slot, cv, koff, W, ue)

                    def scat(sl):
                        for h in range(ng):
                            pltpu.async_copy(rbuf.at[sl, pl.ds(h * EB, EB)],
                                             acc.at[rdst.at[sl, pl.ds(h * EB, EB)]],
                                             ssem[sl], add=True)

                    _sel(2, slot, scat)
                    return ug2 + 1

                return lax.fori_loop(0, (fill + ue - 1) // ue, unit, ug)

            ug = lax.fori_loop(0, nblk, blk_body, jnp.int32(0))
            for dd in range(2):
                @pl.when(ug >= dd + 1)
                def _():
                    _sel(2, (ug - 1 - dd) % 2, s_wait)

        def chunk_body(ch, carry):
            chunk = c * (NCH // NC) + ch
            base = pl.multiple_of(chunk * R + s * stripe, 8)

            def zrow(i, carry2):
                pltpu.sync_copy(zb, acc.at[pl.ds(s * stripe + i * 64, 64)])
                return carry2
            lax.fori_loop(0, stripe // 64, zrow, 0)
            plsc.subcore_barrier()
            scan_and_process(chunk * R)
            plsc.subcore_barrier()
            pltpu.sync_copy(acc.at[pl.ds(s * stripe, stripe)], out_hbm.at[pl.ds(base, stripe)])
            plsc.subcore_barrier()
            return carry

        lax.fori_loop(0, NCH // NC, chunk_body, 0)

    return spmm1


def _pad_zero(a, total):
    return jnp.concatenate([a, jnp.zeros((total - a.shape[0],), a.dtype)])


def _pad_oob(d, s, v, total):
    pad = total - d.shape[0]
    return (jnp.concatenate([d, jnp.full((pad,), jnp.int32(1 << 29))]),
            jnp.concatenate([s, jnp.zeros((pad,), s.dtype)]),
            jnp.concatenate([v, jnp.zeros((pad,), v.dtype)]))


def kernel(X0, X1, B1_row, B1_col, B1_val, L0_row, L0_col, L0_val, L1_row, L1_col, L1_val, W1_0L, W1_0B, W1_0I, W1_1L, W1_1B, W1_1I, W2_0L, W2_0B, W2_0I, W2_1L, W2_1B, W2_1I, W3_0L, W3_0B, W3_0I, W3_1L, W3_1B, W3_1I, Wfc):
    s0 = (_pad_zero(L0_row, T0), _pad_zero(L0_col, T0), _pad_zero(L0_val, T0))
    s1 = (_pad_zero(B1_row, T1), _pad_zero(B1_col, T1), _pad_zero(B1_val, T1))
    uL = _pad_oob(L1_row, L1_col, L1_val, NBL1 * NT * SB)
    uB = _pad_oob(B1_col, B1_row, B1_val, NBB1 * NT * SB)

    Ws = [(W1_0L, W1_0B, W1_0I, W1_1L, W1_1B, W1_1I),
          (W2_0L, W2_0B, W2_0I, W2_1L, W2_1B, W2_1I),
          (W3_0L, W3_0B, W3_0I, W3_1L, W3_1B, W3_1I)]

    h0 = jnp.concatenate([X0, jnp.zeros((N0P - N0, X0.shape[1]), jnp.float32)])
    h1 = X1
    for li, (WL0, WB0, WI0, WL1, WB1, WI1) in enumerate(Ws):
        W = h0.shape[1]
        R = 8192 if W == 128 else 16384
        spmm0 = _make_spmm0(W)
        spmm1L = _make_spmm1(W, R, NBL1)
        spmm1B = _make_spmm1(W, R, NBB1)
        P = spmm0(h0, h1, *s0, *s1)
        SL = spmm1L(h1, *uL)
        SBt = spmm1B(h0, *uB)
        h0 = _combine(P, None, h0, WL0, WB0, WI0, pair=True, blk=2048)
        h1 = _combine(SL, SBt, h1, WL1, WB1, WI1, pair=False, blk=2000)

    out0 = _fc(h0, Wfc, blk=2048)[:N0]
    out1 = _fc(h1, Wfc, blk=2000)
    return (out0, out1)
